# baseline (speedup 1.0000x reference)
import functools

import jax
import jax.numpy as jnp
from jax import lax
from jax.experimental import pallas as pl
from jax.experimental.pallas import tpu as pltpu
from jax.experimental.pallas import tpu_sc as plsc

IGNORE_INDEX = -100
CONFIDENCE = 0.9

_LANES = 16
_CHUNK = 128
_SC_ROWS = 512
_SC_COLCHUNK = 6400


def _loss_body(x_ref, t_ref, oh_ref, rowval_ref, chunk_ref):
    t = t_ref[0, 0, :]

    br, v = x_ref.shape
    zero_col = v + IGNORE_INDEX

    sv = oh_ref[0, 0]
    log_sv = jnp.log(sv)
    sum_oh = sv * (v - 1)
    c_ent = sv * log_sv * (v - 1)

    nchunk = 10
    c = v // nchunk
    m_acc = jnp.max(x_ref[:, :c], axis=1, keepdims=True)
    r_acc = jnp.sum(x_ref[:, :c], axis=1)
    for k in range(1, nchunk):
        xk = x_ref[:, k * c:(k + 1) * c]
        m_acc = jnp.maximum(m_acc, jnp.max(xk, axis=1, keepdims=True))
        r_acc = r_acc + jnp.sum(xk, axis=1)
    m = m_acc

    s = jnp.sum(jnp.exp(x_ref[...] - m), axis=1)
    lse = m[:, 0] + jnp.log(s)

    d = sv * (r_acc - x_ref[:, zero_col])

    for r in range(br):
        start = pl.multiple_of((t_ref[0, 0, r] >> 7) * _CHUNK, _CHUNK)
        chunk_ref[r, :] = x_ref[r, pl.ds(start, _CHUNK)]
    lane = t & (_CHUNK - 1)
    col = lax.broadcasted_iota(jnp.int32, (br, _CHUNK), 1)
    xt = jnp.sum(jnp.where(col == lane[:, None], chunk_ref[...], 0.0), axis=1)

    lp = xt - lse
    row = (c_ent - d + lse * sum_oh
           - jnp.where(t != zero_col, sv * (log_sv - lp), 0.0)
           + CONFIDENCE * (jnp.log(CONFIDENCE) - lp))
    rowval_ref[...] = jnp.where(t != IGNORE_INDEX, row, 0.0).reshape(1, 1, br)


def _make_sc_stream(b, v, nc, ns):
    nw = nc * ns
    rpw = _SC_ROWS // nw
    row0 = b - _SC_ROWS
    ncc = v // _SC_COLCHUNK
    nk = _SC_COLCHUNK // _LANES
    mesh = plsc.VectorSubcoreMesh(core_axis_name="c", subcore_axis_name="s")

    @functools.partial(
        pl.kernel,
        mesh=mesh,
        out_type=[
            jax.ShapeDtypeStruct((_SC_ROWS, _LANES), jnp.float32),
            jax.ShapeDtypeStruct((_SC_ROWS, _LANES), jnp.float32),
            jax.ShapeDtypeStruct((_SC_ROWS, _LANES), jnp.float32),
            jax.ShapeDtypeStruct((_SC_ROWS, _LANES), jnp.float32),
            jax.ShapeDtypeStruct((_SC_ROWS, _LANES), jnp.float32),
        ],
        scratch_types=[
            pltpu.VMEM((8, _SC_COLCHUNK), jnp.float32),
            pltpu.VMEM((rpw,), jnp.int32),
            pltpu.VMEM((rpw, _LANES), jnp.float32),
            pltpu.VMEM((rpw, _LANES), jnp.float32),
            pltpu.VMEM((rpw, _LANES), jnp.float32),
            pltpu.VMEM((rpw, _LANES), jnp.float32),
            pltpu.VMEM((rpw, _LANES), jnp.float32),
        ],
    )
    def stream_kernel(x_hbm, t_hbm, m_hbm, r_hbm, x_out_hbm, e_hbm, z_hbm,
                      buf_v, t_v, m_v, r_v, xt_v, e_v, z_v):
        zero_col = v + IGNORE_INDEX
        wid = lax.axis_index("s") * nc + lax.axis_index("c")
        wrow0 = row0 + wid * rpw
        pltpu.sync_copy(t_hbm.at[pl.ds(wrow0, rpw)], t_v)
        tvec = t_v[...]

        neg_big = jnp.full((_LANES,), -3.0e38, jnp.float32)
        zeros = jnp.zeros((_LANES,), jnp.float32)

        for g in range(rpw // 8):
            for cc in range(ncc):
                pltpu.sync_copy(
                    x_hbm.at[pl.ds(wrow0 + g * 8, 8),
                             pl.ds(cc * _SC_COLCHUNK, _SC_COLCHUNK)], buf_v)
                for r8 in range(8):
                    r = g * 8 + r8
                    t_splat = lax.gather(
                        tvec, jnp.full((_LANES, 1), r, jnp.int32),
                        lax.GatherDimensionNumbers(
                            offset_dims=(), collapsed_slice_dims=(0,),
                            start_index_map=(0,)),
                        slice_sizes=(1,),
                        mode=lax.GatherScatterMode.PROMISE_IN_BOUNDS)

                    def body1(k, carry, r8=r8, cc=cc, t_splat=t_splat):
                        mm, rr, xx, zz = carry
                        data = buf_v[r8, pl.ds(k * _LANES, _LANES)]
                        colid = (lax.iota(jnp.int32, _LANES)
                                 + (cc * _SC_COLCHUNK + k * _LANES))
                        mm = jnp.maximum(mm, data)
                        rr = rr + data
                        xx = xx + jnp.where(colid == t_splat, data, 0.0)
                        zz = zz + jnp.where(colid == zero_col, data, 0.0)
                        return mm, rr, xx, zz

                    if cc == 0:
                        init = (neg_big, zeros, zeros, zeros)
                    else:
                        init = (m_v[r, :], r_v[r, :], xt_v[r, :], z_v[r, :])
                    mm, rr, xx, zz = lax.fori_loop(0, nk, body1, init)
                    m_v[r, :] = mm
                    r_v[r, :] = rr
                    xt_v[r, :] = xx
                    z_v[r, :] = zz

        for g in range(rpw // 8):
            for cc in range(ncc):
                pltpu.sync_copy(
                    x_hbm.at[pl.ds(wrow0 + g * 8, 8),
                             pl.ds(cc * _SC_COLCHUNK, _SC_COLCHUNK)], buf_v)
                for r8 in range(8):
                    r = g * 8 + r8
                    msplat = m_v[r, :]

                    def body2(k, ee, r8=r8, msplat=msplat):
                        data = buf_v[r8, pl.ds(k * _LANES, _LANES)]
                        return ee + jnp.exp(data - msplat)

                    init_e = e_v[r, :] if cc > 0 else zeros
                    ee = lax.fori_loop(0, nk, body2, init_e)
                    e_v[r, :] = ee

        pltpu.sync_copy(m_v, m_hbm.at[pl.ds(wid * rpw, rpw)])
        pltpu.sync_copy(r_v, r_hbm.at[pl.ds(wid * rpw, rpw)])
        pltpu.sync_copy(xt_v, x_out_hbm.at[pl.ds(wid * rpw, rpw)])
        pltpu.sync_copy(e_v, e_hbm.at[pl.ds(wid * rpw, rpw)])
        pltpu.sync_copy(z_v, z_hbm.at[pl.ds(wid * rpw, rpw)])

    return stream_kernel


def _combine_body(rowval_ref, m_ref, r_ref, x_ref, e_ref, z_ref, t_ref,
                  oh_ref, out_ref):
    nsc, _ = m_ref.shape
    v = oh_ref.shape[1]
    zero_col = v + IGNORE_INDEX
    t = t_ref[0, 0, :]

    sv = oh_ref[0, 0]
    log_sv = jnp.log(sv)
    sum_oh = sv * (v - 1)
    c_ent = sv * log_sv * (v - 1)

    m = jnp.max(m_ref[...], axis=1)
    s = jnp.sum(e_ref[...] * jnp.exp(m_ref[...] - m[:, None]), axis=1)
    lse = m + jnp.log(s)
    rsum = jnp.sum(r_ref[...], axis=1)
    xt = jnp.sum(x_ref[...], axis=1)

    xzc = jnp.sum(z_ref[...], axis=1)

    lp = xt - lse
    d = sv * (rsum - xzc)
    row = (c_ent - d + lse * sum_oh
           - jnp.where(t != zero_col, sv * (log_sv - lp), 0.0)
           + CONFIDENCE * (jnp.log(CONFIDENCE) - lp))
    row = jnp.where(t != IGNORE_INDEX, row, 0.0)
    out_ref[...] = (jnp.sum(rowval_ref[...]) + jnp.sum(row)).reshape(1, 1, 1)


@jax.jit
def kernel(output, target, one_hot):
    b, v = output.shape
    br = 128
    tc_rows = b - _SC_ROWS
    nb = tc_rows // br
    target3 = target[:tc_rows].reshape(nb, 1, br)

    rowvals = pl.pallas_call(
        _loss_body,
        grid=(nb,),
        in_specs=[
            pl.BlockSpec((br, v), lambda i: (i, 0)),
            pl.BlockSpec((1, 1, br), lambda i: (i, 0, 0)),
            pl.BlockSpec((1, v), lambda i: (0, 0)),
        ],
        out_specs=pl.BlockSpec((1, 1, br), lambda i: (i, 0, 0)),
        out_shape=jax.ShapeDtypeStruct((nb, 1, br), jnp.float32),
        scratch_shapes=[pltpu.VMEM((br, _CHUNK), jnp.float32)],
    )(output[:tc_rows], target3, one_hot)

    info = plsc.get_sparse_core_info()
    m16, r16, x16, e16, z16 = _make_sc_stream(b, v, info.num_cores,
                                              info.num_subcores)(output, target)

    tsc3 = target[tc_rows:].reshape(1, 1, _SC_ROWS)
    total = pl.pallas_call(
        _combine_body,
        in_specs=[
            pl.BlockSpec((nb, 1, br), lambda: (0, 0, 0)),
            pl.BlockSpec((_SC_ROWS, _LANES), lambda: (0, 0)),
            pl.BlockSpec((_SC_ROWS, _LANES), lambda: (0, 0)),
            pl.BlockSpec((_SC_ROWS, _LANES), lambda: (0, 0)),
            pl.BlockSpec((_SC_ROWS, _LANES), lambda: (0, 0)),
            pl.BlockSpec((_SC_ROWS, _LANES), lambda: (0, 0)),
            pl.BlockSpec((1, 1, _SC_ROWS), lambda: (0, 0, 0)),
            pl.BlockSpec((1, v), lambda: (0, 0)),
        ],
        out_specs=pl.BlockSpec((1, 1, 1), lambda: (0, 0, 0)),
        out_shape=jax.ShapeDtypeStruct((1, 1, 1), jnp.float32),
    )(rowvals, m16, r16, x16, e16, z16, tsc3, one_hot)

    return total[0, 0, 0] / b


# speedup vs baseline: 2.6549x; 2.6549x over previous
import functools

import jax
import jax.numpy as jnp
from jax import lax
from jax.experimental import pallas as pl
from jax.experimental.pallas import tpu as pltpu
from jax.experimental.pallas import tpu_sc as plsc

IGNORE_INDEX = -100
CONFIDENCE = 0.9

_LANES = 16
_CHUNK = 128


def _loss_body(x_ref, t_ref, oh_ref, rowval_ref, chunk_ref):
    t = t_ref[0, 0, :]

    br, v = x_ref.shape
    zero_col = v + IGNORE_INDEX

    sv = oh_ref[0, 0]
    log_sv = jnp.log(sv)
    sum_oh = sv * (v - 1)
    c_ent = sv * log_sv * (v - 1)

    nchunk = 10
    c = v // nchunk
    m_acc = jnp.max(x_ref[:, :c], axis=1, keepdims=True)
    r_acc = jnp.sum(x_ref[:, :c], axis=1)
    for k in range(1, nchunk):
        xk = x_ref[:, k * c:(k + 1) * c]
        m_acc = jnp.maximum(m_acc, jnp.max(xk, axis=1, keepdims=True))
        r_acc = r_acc + jnp.sum(xk, axis=1)
    m = m_acc

    s = jnp.sum(jnp.exp(x_ref[...] - m), axis=1)
    lse = m[:, 0] + jnp.log(s)

    d = sv * (r_acc - x_ref[:, zero_col])

    for r in range(br):
        start = pl.multiple_of((t_ref[0, 0, r] >> 7) * _CHUNK, _CHUNK)
        chunk_ref[r, :] = x_ref[r, pl.ds(start, _CHUNK)]
    lane = t & (_CHUNK - 1)
    col = lax.broadcasted_iota(jnp.int32, (br, _CHUNK), 1)
    xt = jnp.sum(jnp.where(col == lane[:, None], chunk_ref[...], 0.0), axis=1)

    lp = xt - lse
    row = (c_ent - d + lse * sum_oh
           - jnp.where(t != zero_col, sv * (log_sv - lp), 0.0)
           + CONFIDENCE * (jnp.log(CONFIDENCE) - lp))
    rowval_ref[...] = jnp.where(t != IGNORE_INDEX, row, 0.0).reshape(1, 1, br)


def _make_sc_reduce(b, nc, ns):
    nw = nc * ns
    bpw = b // nw
    mesh = plsc.VectorSubcoreMesh(core_axis_name="c", subcore_axis_name="s")

    @functools.partial(
        pl.kernel,
        mesh=mesh,
        out_type=jax.ShapeDtypeStruct((nw, _LANES), jnp.float32),
        scratch_types=[
            pltpu.VMEM((bpw,), jnp.float32),
            pltpu.VMEM((_LANES,), jnp.float32),
        ],
    )
    def reduce_kernel(rowval_hbm, out_hbm, rowval_v, acc_v):
        wid = lax.axis_index("s") * nc + lax.axis_index("c")
        base = wid * bpw
        pltpu.sync_copy(rowval_hbm.at[pl.ds(base, bpw)], rowval_v)
        acc = jnp.zeros((_LANES,), jnp.float32)
        for j in range(bpw // _LANES):
            acc = acc + rowval_v[pl.ds(j * _LANES, _LANES)]
        acc_v[...] = acc
        pltpu.sync_copy(acc_v, out_hbm.at[wid])

    return reduce_kernel


@jax.jit
def kernel(output, target, one_hot):
    b, v = output.shape
    br = 128
    nb = b // br
    target3 = target.reshape(nb, 1, br)

    rowvals = pl.pallas_call(
        _loss_body,
        grid=(nb,),
        in_specs=[
            pl.BlockSpec((br, v), lambda i: (i, 0)),
            pl.BlockSpec((1, 1, br), lambda i: (i, 0, 0)),
            pl.BlockSpec((1, v), lambda i: (0, 0)),
        ],
        out_specs=pl.BlockSpec((1, 1, br), lambda i: (i, 0, 0)),
        out_shape=jax.ShapeDtypeStruct((nb, 1, br), jnp.float32),
        scratch_shapes=[pltpu.VMEM((br, _CHUNK), jnp.float32)],
    )(output, target3, one_hot)

    info = plsc.get_sparse_core_info()
    partials = _make_sc_reduce(b, info.num_cores, info.num_subcores)(
        rowvals.reshape(b))

    return jnp.sum(partials) / b
